# baseline (speedup 1.0000x reference)
import functools

import jax
import jax.numpy as jnp
from jax import lax
from jax.experimental import pallas as pl
from jax.experimental.pallas import tpu as pltpu
from jax.experimental.pallas import tpu_sc as plsc

VOCAB = 50257
N_EMBD = 768
BLOCK = 2048
B, T = 4, 2048

NC, NS, L = 2, 16, 16
NW = NC * NS
TPW = T // NW
VECS = N_EMBD // L
G = 16
CHUNKS = TPW // G
ROWS = B * G

_mesh = plsc.VectorSubcoreMesh(core_axis_name="c", subcore_axis_name="s")


@functools.partial(
    pl.kernel,
    mesh=_mesh,
    out_type=jax.ShapeDtypeStruct((B * T, N_EMBD), jnp.float32),
    scratch_types=[
        pltpu.VMEM((B * TPW,), jnp.int32),
        pltpu.VMEM((2, ROWS, N_EMBD), jnp.float32),
        pltpu.VMEM((2, G, N_EMBD), jnp.float32),
    ] + [pltpu.SemaphoreType.DMA] * 6,
)
def _embed(idx_hbm, wte_hbm, wpe_hbm, out_hbm,
           idx_v, tok_v, wpe_v, g0, g1, w0, w1, s0, s1):
    gsem, wsem, ssem = (g0, g1), (w0, w1), (s0, s1)
    wid = lax.axis_index("s") * NC + lax.axis_index("c")
    t0 = wid * TPW

    def issue(c, slot):
        hw = pltpu.async_copy(wpe_hbm.at[pl.ds(t0 + c * G, G)],
                              wpe_v.at[slot], wsem[slot])
        hg = pltpu.async_copy(
            wte_hbm.at[idx_v.at[pl.ds(c * ROWS, ROWS)]],
            tok_v.at[slot], gsem[slot])
        return hg, hw

    pltpu.sync_copy(idx_hbm.at[pl.ds(wid * B * TPW, B * TPW)], idx_v)
    hin, hst = {}, {}
    hin[0] = issue(0, 0)
    hin[1] = issue(1, 1)

    for c in range(CHUNKS):
        slot = c % 2
        hin[c][0].wait()
        hin[c][1].wait()

        @plsc.parallel_loop(0, G, 1, unroll=2)
        def row_add(tt, slot=slot):
            for j in range(VECS):
                sl = pl.ds(j * L, L)
                w = wpe_v[slot, tt, sl]
                for b in range(B):
                    r = b * G + tt
                    tok_v[slot, r, sl] = tok_v[slot, r, sl] + w
        hst[c] = [
            pltpu.async_copy(tok_v.at[slot].at[pl.ds(b * G, G)],
                             out_hbm.at[pl.ds(b * T + t0 + c * G, G)],
                             ssem[slot])
            for b in range(B)
        ]
        if c + 2 < CHUNKS:
            for h in hst[c]:
                h.wait()
            hin[c + 2] = issue(c + 2, slot)

    for c in (CHUNKS - 2, CHUNKS - 1):
        for h in hst[c]:
            h.wait()


def kernel(idx, wte, wpe):
    idx_r = jnp.transpose(
        idx.astype(jnp.int32).reshape(B, NW, CHUNKS, G), (1, 2, 0, 3)
    ).reshape(-1)
    flat = _embed(idx_r, wte, wpe)
    return flat.reshape(B, T, N_EMBD)


# speedup vs baseline: 1.8742x; 1.8742x over previous
import functools

import jax
import jax.numpy as jnp
from jax import lax
from jax.experimental import pallas as pl
from jax.experimental.pallas import tpu as pltpu
from jax.experimental.pallas import tpu_sc as plsc

VOCAB = 50257
N_EMBD = 768
BLOCK = 2048
B, T = 4, 2048

NC, NS, L = 2, 16, 16
NW = NC * NS
TPW = T // NW
VECS = N_EMBD // L
G = 16
CHUNKS = TPW // G
ROWS = B * G

_mesh = plsc.VectorSubcoreMesh(core_axis_name="c", subcore_axis_name="s")


@functools.partial(
    pl.kernel,
    mesh=_mesh,
    out_type=jax.ShapeDtypeStruct((B * T, N_EMBD), jnp.float32),
    scratch_types=[
        pltpu.VMEM((B * TPW,), jnp.int32),
        pltpu.VMEM((2, ROWS, N_EMBD), jnp.float32),
        pltpu.VMEM((2, G, N_EMBD), jnp.float32),
    ] + [pltpu.SemaphoreType.DMA] * 6,
)
def _embed(idx_hbm, wte_hbm, wpe_hbm, out_hbm,
           idx_v, tok_v, wpe_v, g0, g1, w0, w1, s0, s1):
    gsem, wsem, ssem = (g0, g1), (w0, w1), (s0, s1)
    wid = lax.axis_index("s") * NC + lax.axis_index("c")
    t0 = wid * TPW

    def issue(c, slot):
        hw = pltpu.async_copy(wpe_hbm.at[pl.ds(t0 + c * G, G)],
                              wpe_v.at[slot], wsem[slot])
        hg = pltpu.async_copy(
            wte_hbm.at[idx_v.at[pl.ds(c * ROWS, ROWS)]],
            tok_v.at[slot], gsem[slot])
        return hg, hw

    pltpu.sync_copy(idx_hbm.at[pl.ds(wid * B * TPW, B * TPW)], idx_v)
    hin, hst = {}, {}
    hin[0] = issue(0, 0)
    hin[1] = issue(1, 1)

    for c in range(CHUNKS):
        slot = c % 2
        hin[c][0].wait()
        hin[c][1].wait()

        @plsc.parallel_loop(0, VECS, 1, unroll=1)
        def col_add(j, slot=slot):
            sl = pl.ds(j * L, L)
            for tt in range(G):
                w = wpe_v[slot, tt, sl]
                for b in range(B):
                    r = b * G + tt
                    tok_v[slot, r, sl] = tok_v[slot, r, sl] + w
        hst[c] = [
            pltpu.async_copy(tok_v.at[slot].at[pl.ds(b * G, G)],
                             out_hbm.at[pl.ds(b * T + t0 + c * G, G)],
                             ssem[slot])
            for b in range(B)
        ]
        if c + 2 < CHUNKS:
            for h in hst[c]:
                h.wait()
            hin[c + 2] = issue(c + 2, slot)

    for c in (CHUNKS - 2, CHUNKS - 1):
        for h in hst[c]:
            h.wait()


def kernel(idx, wte, wpe):
    idx_r = jnp.transpose(
        idx.astype(jnp.int32).reshape(B, NW, CHUNKS, G), (1, 2, 0, 3)
    ).reshape(-1)
    flat = _embed(idx_r, wte, wpe)
    return flat.reshape(B, T, N_EMBD)


# speedup vs baseline: 1.9337x; 1.0318x over previous
import functools

import jax
import jax.numpy as jnp
from jax import lax
from jax.experimental import pallas as pl
from jax.experimental.pallas import tpu as pltpu
from jax.experimental.pallas import tpu_sc as plsc

VOCAB = 50257
N_EMBD = 768
BLOCK = 2048
B, T = 4, 2048

NC, NS, L = 2, 16, 16
NW = NC * NS
TPW = T // NW
VECS = N_EMBD // L
G = 16
CHUNKS = TPW // G
ROWS = B * G

_mesh = plsc.VectorSubcoreMesh(core_axis_name="c", subcore_axis_name="s")


@functools.partial(
    pl.kernel,
    mesh=_mesh,
    out_type=jax.ShapeDtypeStruct((B * T, N_EMBD), jnp.float32),
    scratch_types=[
        pltpu.VMEM((B * TPW,), jnp.int32),
        pltpu.VMEM((2, ROWS, N_EMBD), jnp.float32),
        pltpu.VMEM((2, G, N_EMBD), jnp.float32),
    ] + [pltpu.SemaphoreType.DMA] * 6,
)
def _embed(idx_hbm, wte_hbm, wpe_hbm, out_hbm,
           idx_v, tok_v, wpe_v, g0, g1, w0, w1, s0, s1):
    gsem, wsem, ssem = (g0, g1), (w0, w1), (s0, s1)
    wid = lax.axis_index("s") * NC + lax.axis_index("c")
    t0 = wid * TPW

    def issue(c, slot):
        pltpu.async_copy(wpe_hbm.at[pl.ds(t0 + c * G, G)],
                         wpe_v.at[slot], wsem[slot])
        pltpu.async_copy(wte_hbm.at[idx_v.at[pl.ds(c * ROWS, ROWS)]],
                         tok_v.at[slot], gsem[slot])

    def wait_in(slot):
        pltpu.make_async_copy(wpe_hbm.at[pl.ds(0, G)],
                              wpe_v.at[slot], wsem[slot]).wait()
        pltpu.make_async_copy(wte_hbm.at[idx_v.at[pl.ds(0, ROWS)]],
                              tok_v.at[slot], gsem[slot]).wait()

    def issue_stores(c, slot):
        for b in range(B):
            pltpu.async_copy(tok_v.at[slot].at[pl.ds(b * G, G)],
                             out_hbm.at[pl.ds(b * T + t0 + c * G, G)],
                             ssem[slot])

    def wait_stores(slot):
        for _ in range(B):
            pltpu.make_async_copy(tok_v.at[slot].at[pl.ds(0, G)],
                                  out_hbm.at[pl.ds(0, G)], ssem[slot]).wait()

    def add(slot):
        @plsc.parallel_loop(0, VECS, 1, unroll=1)
        def col_add(j):
            sl = pl.ds(j * L, L)
            for tt in range(G):
                w = wpe_v[slot, tt, sl]
                for b in range(B):
                    r = b * G + tt
                    tok_v[slot, r, sl] = tok_v[slot, r, sl] + w

    pltpu.sync_copy(idx_hbm.at[pl.ds(wid * B * TPW, B * TPW)], idx_v)
    issue(0, 0)
    issue(1, 1)

    def pair_body(p, carry):
        for slot in range(2):
            c = 2 * p + slot
            wait_in(slot)
            add(slot)
            issue_stores(c, slot)

            @pl.when(c + 2 < CHUNKS)
            def _():
                wait_stores(slot)
                issue(c + 2, slot)
        return carry

    lax.fori_loop(0, CHUNKS // 2, pair_body, 0)
    wait_stores(0)
    wait_stores(1)


def kernel(idx, wte, wpe):
    idx_r = jnp.transpose(
        idx.astype(jnp.int32).reshape(B, NW, CHUNKS, G), (1, 2, 0, 3)
    ).reshape(-1)
    flat = _embed(idx_r, wte, wpe)
    return flat.reshape(B, T, N_EMBD)


# speedup vs baseline: 2.0712x; 1.0711x over previous
import functools

import jax
import jax.numpy as jnp
from jax import lax
from jax.experimental import pallas as pl
from jax.experimental.pallas import tpu as pltpu
from jax.experimental.pallas import tpu_sc as plsc

VOCAB = 50257
N_EMBD = 768
BLOCK = 2048
B, T = 4, 2048

NC, NS, L = 2, 16, 16
NW = NC * NS
TPW = T // NW
VECS = N_EMBD // L
G = 8
CHUNKS = TPW // G
ROWS = B * G
SLOTS = 4

_mesh = plsc.VectorSubcoreMesh(core_axis_name="c", subcore_axis_name="s")


@functools.partial(
    pl.kernel,
    mesh=_mesh,
    out_type=jax.ShapeDtypeStruct((B * T, N_EMBD), jnp.float32),
    scratch_types=[
        pltpu.VMEM((B * TPW,), jnp.int32),
        pltpu.VMEM((SLOTS, ROWS, N_EMBD), jnp.float32),
        pltpu.VMEM((SLOTS, G, N_EMBD), jnp.float32),
    ] + [pltpu.SemaphoreType.DMA] * (3 * SLOTS),
)
def _embed(idx_hbm, wte_hbm, wpe_hbm, out_hbm, idx_v, tok_v, wpe_v, *sems):
    gsem, wsem, ssem = sems[:SLOTS], sems[SLOTS:2 * SLOTS], sems[2 * SLOTS:]
    wid = lax.axis_index("s") * NC + lax.axis_index("c")
    t0 = wid * TPW

    def issue(c, slot):
        pltpu.async_copy(wpe_hbm.at[pl.ds(t0 + c * G, G)],
                         wpe_v.at[slot], wsem[slot])
        pltpu.async_copy(wte_hbm.at[idx_v.at[pl.ds(c * ROWS, ROWS)]],
                         tok_v.at[slot], gsem[slot])

    def wait_in(slot):
        pltpu.make_async_copy(wpe_hbm.at[pl.ds(0, G)],
                              wpe_v.at[slot], wsem[slot]).wait()
        pltpu.make_async_copy(wte_hbm.at[idx_v.at[pl.ds(0, ROWS)]],
                              tok_v.at[slot], gsem[slot]).wait()

    def issue_stores(c, slot):
        for b in range(B):
            pltpu.async_copy(tok_v.at[slot].at[pl.ds(b * G, G)],
                             out_hbm.at[pl.ds(b * T + t0 + c * G, G)],
                             ssem[slot])

    def wait_stores(slot):
        for _ in range(B):
            pltpu.make_async_copy(tok_v.at[slot].at[pl.ds(0, G)],
                                  out_hbm.at[pl.ds(0, G)], ssem[slot]).wait()

    def add(slot):
        @plsc.parallel_loop(0, VECS, 1, unroll=1)
        def col_add(j):
            sl = pl.ds(j * L, L)
            for tt in range(G):
                w = wpe_v[slot, tt, sl]
                for b in range(B):
                    r = b * G + tt
                    tok_v[slot, r, sl] = tok_v[slot, r, sl] + w

    pltpu.sync_copy(idx_hbm.at[pl.ds(wid * B * TPW, B * TPW)], idx_v)
    for slot in range(SLOTS):
        issue(slot, slot)

    def cycle_body(q, carry):
        for slot in range(SLOTS):
            c = SLOTS * q + slot
            wait_in(slot)
            add(slot)
            issue_stores(c, slot)

            @pl.when(c + SLOTS < CHUNKS)
            def _():
                wait_stores(slot)
                issue(c + SLOTS, slot)
        return carry

    lax.fori_loop(0, CHUNKS // SLOTS, cycle_body, 0)
    for slot in range(SLOTS):
        wait_stores(slot)


def kernel(idx, wte, wpe):
    idx_r = jnp.transpose(
        idx.astype(jnp.int32).reshape(B, NW, CHUNKS, G), (1, 2, 0, 3)
    ).reshape(-1)
    flat = _embed(idx_r, wte, wpe)
    return flat.reshape(B, T, N_EMBD)
